# baseline (speedup 1.0000x reference)
import jax
import jax.numpy as jnp
from jax import lax
from jax.experimental import pallas as pl
from jax.experimental.pallas import tpu as pltpu
from jax.experimental.pallas import tpu_sc as plsc

_B = 4
_L = 8192
_D = 1024
_BLK = 256

_info = plsc.get_sparse_core_info()
_NC = _info.num_cores
_NS = _info.num_subcores
_NW = _NC * _NS
_ROWS_PER_W = _L // _NW
_CHUNK = 64
_NCHUNK = _ROWS_PER_W // _CHUNK

_B_SC = 2
_B_TC = _B - _B_SC


def _sc_body(table_hbm, out_hbm, buf, sem):
    wid = lax.axis_index("s") * _NC + lax.axis_index("c")
    base = wid * _ROWS_PER_W
    for i in range(_NCHUNK):
        row0 = base + i * _CHUNK
        pltpu.async_copy(table_hbm.at[pl.ds(row0, _CHUNK)], buf, sem).wait()
        for b in range(_B_SC):
            pltpu.async_copy(buf, out_hbm.at[b, pl.ds(row0, _CHUNK)], sem).wait()


def _tc_body(table_ref, out_ref):
    out_ref[...] = jnp.broadcast_to(table_ref[...][None], (_B_TC, _BLK, _D))


def kernel(x, table):
    del x
    mesh = plsc.VectorSubcoreMesh(core_axis_name="c", subcore_axis_name="s")
    sc_run = pl.kernel(
        _sc_body,
        mesh=mesh,
        out_type=jax.ShapeDtypeStruct((_B_SC, _L, _D), jnp.float32),
        scratch_types=[
            pltpu.VMEM((_CHUNK, _D), jnp.float32),
            pltpu.SemaphoreType.DMA,
        ],
    )
    sc_half = sc_run(table)
    tc_half = pl.pallas_call(
        _tc_body,
        grid=(_L // _BLK,),
        in_specs=[pl.BlockSpec((_BLK, _D), lambda j: (j, 0))],
        out_specs=pl.BlockSpec((_B_TC, _BLK, _D), lambda j: (0, j, 0)),
        out_shape=jax.ShapeDtypeStruct((_B_TC, _L, _D), jnp.float32),
    )(table)
    return jnp.concatenate([tc_half, sc_half], axis=0)


# speedup vs baseline: 2.2756x; 2.2756x over previous
import jax
import jax.numpy as jnp
from jax import lax
from jax.experimental import pallas as pl
from jax.experimental.pallas import tpu as pltpu
from jax.experimental.pallas import tpu_sc as plsc

_B = 4
_L = 8192
_D = 1024

_info = plsc.get_sparse_core_info()
_NC = _info.num_cores
_NS = _info.num_subcores
_NW = _NC * _NS
_ROWS_PER_W = _L // _NW
_CHUNK = 64
_NCHUNK = _ROWS_PER_W // _CHUNK


def _copy_body(table_hbm, out_hbm, buf, ld_sem, st_sem):
    wid = lax.axis_index("s") * _NC + lax.axis_index("c")
    base = wid * _ROWS_PER_W
    pending = []
    for i in range(_NCHUNK):
        row0 = base + i * _CHUNK
        for c in pending:
            c.wait()
        pltpu.async_copy(table_hbm.at[pl.ds(row0, _CHUNK)], buf, ld_sem).wait()
        pending = [
            pltpu.async_copy(buf, out_hbm.at[b, pl.ds(row0, _CHUNK)], st_sem)
            for b in range(_B)
        ]
    for c in pending:
        c.wait()


def kernel(x, table):
    del x
    mesh = plsc.VectorSubcoreMesh(core_axis_name="c", subcore_axis_name="s")
    run = pl.kernel(
        _copy_body,
        mesh=mesh,
        out_type=jax.ShapeDtypeStruct((_B, _L, _D), jnp.float32),
        scratch_types=[
            pltpu.VMEM((_CHUNK, _D), jnp.float32),
            pltpu.SemaphoreType.DMA,
            pltpu.SemaphoreType.DMA,
        ],
    )
    return run(table)
